# Initial kernel scaffold; baseline (speedup 1.0000x reference)
import jax
import jax.numpy as jnp
from jax.experimental import pallas as pl


def kernel(x, tables):
    raise NotImplementedError("write your pallas kernel here")


# speedup vs baseline: 1.1833x; 1.1833x over previous
import jax
import jax.numpy as jnp
from jax import lax
from jax.experimental import pallas as pl
from jax.experimental.pallas import tpu as pltpu
from jax.experimental.pallas import tpu_sc as plsc

N_FIELDS = 26
VOCAB = 100000
HIDDEN = 32
BATCH = 16384

NUM_CORES = 2
NUM_SUBCORES = 16
NW = NUM_CORES * NUM_SUBCORES
BPW = BATCH // NW
CHUNK = 64
NCHUNK = BPW // CHUNK
ROWS = CHUNK * N_FIELDS
IDX_PER_GATHER = 128
GPC = ROWS // IDX_PER_GATHER
IDX_ROWS_PW = BPW * N_FIELDS // IDX_PER_GATHER


def _body(idx_hbm, tab_hbm, out_hbm, idx_v, rows_v, out_v, sem):
    wid = lax.axis_index("s") * NUM_CORES + lax.axis_index("c")
    pltpu.sync_copy(idx_hbm.at[wid], idx_v)

    def chunk_body(t, carry):
        handles = []
        for j in range(GPC):
            handles.append(pltpu.async_copy(
                tab_hbm.at[idx_v.at[t * GPC + j]],
                rows_v.at[pl.ds(j * IDX_PER_GATHER, IDX_PER_GATHER)],
                sem))
        for h in handles:
            h.wait()

        def elem_body(c, carry2):
            base = c * N_FIELDS
            a0 = rows_v[base, 0:16]
            a1 = rows_v[base, 16:32]
            for i in range(1, N_FIELDS):
                a0 = a0 + rows_v[base + i, 0:16]
                a1 = a1 + rows_v[base + i, 16:32]
            out_v[c, 0:16] = a0
            out_v[c, 16:32] = a1
            return carry2

        lax.fori_loop(0, CHUNK, elem_body, 0)
        pltpu.sync_copy(out_v, out_hbm.at[pl.ds(wid * BPW + t * CHUNK, CHUNK)])
        return carry

    lax.fori_loop(0, NCHUNK, chunk_body, 0)


_mesh = plsc.VectorSubcoreMesh(core_axis_name="c", subcore_axis_name="s")

_sc_call = pl.kernel(
    _body,
    out_type=jax.ShapeDtypeStruct((BATCH, HIDDEN), jnp.float32),
    mesh=_mesh,
    scratch_types=[
        pltpu.VMEM((IDX_ROWS_PW, IDX_PER_GATHER), jnp.int32),
        pltpu.VMEM((ROWS, HIDDEN), jnp.float32),
        pltpu.VMEM((CHUNK, HIDDEN), jnp.float32),
        pltpu.SemaphoreType.DMA,
    ],
    compiler_params=pltpu.CompilerParams(use_tc_tiling_on_sc=False),
)


def kernel(x, tables):
    x = x.astype(jnp.int32)
    offs = (jnp.arange(N_FIELDS, dtype=jnp.int32) * VOCAB)[:, None]
    flat_idx = (x + offs).T.reshape(NW, IDX_ROWS_PW, IDX_PER_GATHER)
    tab_flat = tables.reshape(N_FIELDS * VOCAB, HIDDEN)
    return _sc_call(flat_idx, tab_flat)


# speedup vs baseline: 1.1859x; 1.0022x over previous
import jax
import jax.numpy as jnp
from jax import lax
from jax.experimental import pallas as pl
from jax.experimental.pallas import tpu as pltpu
from jax.experimental.pallas import tpu_sc as plsc

N_FIELDS = 26
VOCAB = 100000
HIDDEN = 32
BATCH = 16384

NUM_CORES = 2
NUM_SUBCORES = 16
NW = NUM_CORES * NUM_SUBCORES
BPW = BATCH // NW
CHUNK = 64
NCHUNK = BPW // CHUNK
ROWS = CHUNK * N_FIELDS
LANES = 16


def _body(x_hbm, tab_hbm, out_hbm, idx_v, rows_v, out_v, sem):
    wid = lax.axis_index("s") * NUM_CORES + lax.axis_index("c")

    def chunk_body(t, carry):
        pltpu.sync_copy(x_hbm.at[:, wid, t], idx_v)
        for i in range(N_FIELDS):
            for k in range(CHUNK // LANES):
                sl = pl.ds(k * LANES, LANES)
                idx_v[i, sl] = idx_v[i, sl] + jnp.int32(i * VOCAB)
        handles = []
        for i in range(N_FIELDS):
            handles.append(pltpu.async_copy(
                tab_hbm.at[idx_v.at[i]],
                rows_v.at[pl.ds(i * CHUNK, CHUNK)],
                sem))
        for h in handles:
            h.wait()

        def elem_body(c, carry2):
            a0 = rows_v[c, 0:16]
            a1 = rows_v[c, 16:32]
            for i in range(1, N_FIELDS):
                a0 = a0 + rows_v[i * CHUNK + c, 0:16]
                a1 = a1 + rows_v[i * CHUNK + c, 16:32]
            out_v[c, 0:16] = a0
            out_v[c, 16:32] = a1
            return carry2

        lax.fori_loop(0, CHUNK, elem_body, 0)
        pltpu.sync_copy(out_v, out_hbm.at[pl.ds(wid * BPW + t * CHUNK, CHUNK)])
        return carry

    lax.fori_loop(0, NCHUNK, chunk_body, 0)


_mesh = plsc.VectorSubcoreMesh(core_axis_name="c", subcore_axis_name="s")

_sc_call = pl.kernel(
    _body,
    out_type=jax.ShapeDtypeStruct((BATCH, HIDDEN), jnp.float32),
    mesh=_mesh,
    scratch_types=[
        pltpu.VMEM((N_FIELDS, CHUNK), jnp.int32),
        pltpu.VMEM((ROWS, HIDDEN), jnp.float32),
        pltpu.VMEM((CHUNK, HIDDEN), jnp.float32),
        pltpu.SemaphoreType.DMA,
    ],
    compiler_params=pltpu.CompilerParams(use_tc_tiling_on_sc=False),
)


def kernel(x, tables):
    x4 = x.astype(jnp.int32).reshape(N_FIELDS, NW, NCHUNK, CHUNK)
    tab_flat = tables.reshape(N_FIELDS * VOCAB, HIDDEN)
    return _sc_call(x4, tab_flat)


# speedup vs baseline: 1.1987x; 1.0108x over previous
import jax
import jax.numpy as jnp
from jax import lax
from jax.experimental import pallas as pl
from jax.experimental.pallas import tpu as pltpu
from jax.experimental.pallas import tpu_sc as plsc

N_FIELDS = 26
VOCAB = 100000
HIDDEN = 32
BATCH = 16384

NUM_CORES = 2
NUM_SUBCORES = 16
NW = NUM_CORES * NUM_SUBCORES
BPW = BATCH // NW
CHUNK = 64
NCHUNK = BPW // CHUNK
ROWS = CHUNK * N_FIELDS
IWIDTH = 128
NSTREAM = ROWS // IWIDTH
LANES = 16


def _body(x_hbm, tab_hbm, out_hbm, idx_v, rows_v, out_v, sem0, sem1):
    wid = lax.axis_index("s") * NUM_CORES + lax.axis_index("c")
    sems = [sem0, sem1]

    def load_and_fire(t, b):
        pltpu.sync_copy(x_hbm.at[wid, t], idx_v.at[b])
        for r in range(NSTREAM):
            for k in range(IWIDTH // LANES):
                f = (r * IWIDTH + k * LANES) // CHUNK
                sl = pl.ds(k * LANES, LANES)
                idx_v[b, r, sl] = idx_v[b, r, sl] + jnp.int32(f * VOCAB)
        return [
            pltpu.async_copy(
                tab_hbm.at[idx_v.at[b, r]],
                rows_v.at[b, pl.ds(r * IWIDTH, IWIDTH)],
                sems[b])
            for r in range(NSTREAM)
        ]

    def accumulate(b):
        def elem_body(c, carry):
            a0 = rows_v[b, c, 0:16]
            a1 = rows_v[b, c, 16:32]
            for i in range(1, N_FIELDS):
                a0 = a0 + rows_v[b, i * CHUNK + c, 0:16]
                a1 = a1 + rows_v[b, i * CHUNK + c, 16:32]
            out_v[b, c, 0:16] = a0
            out_v[b, c, 16:32] = a1
            return carry

        lax.fori_loop(0, CHUNK, elem_body, 0)

    handles = load_and_fire(0, 0)
    for t in range(NCHUNK):
        b = t % 2
        nxt = None
        if t + 1 < NCHUNK:
            nxt = load_and_fire(t + 1, (t + 1) % 2)
        for h in handles:
            h.wait()
        accumulate(b)
        pltpu.sync_copy(out_v.at[b],
                        out_hbm.at[pl.ds(wid * BPW + t * CHUNK, CHUNK)])
        handles = nxt


_mesh = plsc.VectorSubcoreMesh(core_axis_name="c", subcore_axis_name="s")

_sc_call = pl.kernel(
    _body,
    out_type=jax.ShapeDtypeStruct((BATCH, HIDDEN), jnp.float32),
    mesh=_mesh,
    scratch_types=[
        pltpu.VMEM((2, NSTREAM, IWIDTH), jnp.int32),
        pltpu.VMEM((2, ROWS, HIDDEN), jnp.float32),
        pltpu.VMEM((2, CHUNK, HIDDEN), jnp.float32),
        pltpu.SemaphoreType.DMA,
        pltpu.SemaphoreType.DMA,
    ],
    compiler_params=pltpu.CompilerParams(use_tc_tiling_on_sc=False),
)


def kernel(x, tables):
    x4 = (x.astype(jnp.int32)
          .reshape(N_FIELDS, NW, NCHUNK, CHUNK)
          .transpose(1, 2, 0, 3)
          .reshape(NW, NCHUNK, NSTREAM, IWIDTH))
    tab_flat = tables.reshape(N_FIELDS * VOCAB, HIDDEN)
    return _sc_call(x4, tab_flat)
